# baseline (speedup 1.0000x reference)
import functools

import jax
import jax.numpy as jnp
from jax import lax
from jax.experimental import pallas as pl
from jax.experimental.pallas import tpu as pltpu
from jax.experimental.pallas import tpu_sc as plsc

N = 10000
R = 8
H = 128
NB = 6
NE = 320000

NC = 2
NS = 16
L = 16

EW = NE // (NC * NS)
CW = NE // NS
SK = 2000
CK = 80
CJ = SK // CK


def _mesh():
    return plsc.VectorSubcoreMesh(core_axis_name="c", subcore_axis_name="s",
                                  num_cores=NC, num_subcores=NS)


@functools.cache
def _make_sc_edge_weights():
    @functools.partial(
        pl.kernel,
        out_type=jax.ShapeDtypeStruct((NE,), jnp.float32),
        mesh=_mesh(),
        scratch_types=[
            pltpu.VMEM_SHARED((N * R,), jnp.float32),
            pltpu.VMEM((N * R,), jnp.float32),
            pltpu.VMEM((SK,), jnp.int32),
            pltpu.VMEM((SK,), jnp.int32),
            pltpu.VMEM((CJ, CK), jnp.int32),
            pltpu.VMEM((CK,), jnp.float32),
            pltpu.VMEM((SK,), jnp.float32),
            pltpu.VMEM((5008,), jnp.float32),
        ],
        compiler_params=pltpu.CompilerParams(needs_layout_passes=False),
    )
    def sc_edge_weights(dst_hbm, rel_hbm, w_hbm, cs, cl, dv, rv, ci2, ones,
                        wv, zb):
        cid = lax.axis_index("c")
        sid = lax.axis_index("s")
        wid = sid * NC + cid

        for t in range(CK // L):
            ones[pl.ds(t * L, L)] = jnp.ones((L,), jnp.float32)
        zeros = jnp.zeros((L,), jnp.float32)

        def zfill(i, carry):
            zb[pl.ds(i * L, L)] = zeros
            return carry
        lax.fori_loop(0, 313, zfill, None)
        zn = N * R // NS
        pltpu.sync_copy(zb.at[pl.ds(0, zn)], cs.at[pl.ds(sid * zn, zn)])
        plsc.subcore_barrier()

        def count_super(j, carry):
            base = sid * CW + j * SK
            pltpu.sync_copy(dst_hbm.at[pl.ds(base, SK)], dv)
            pltpu.sync_copy(rel_hbm.at[pl.ds(base, SK)], rv)

            def build_row(ci, c2):
                eb = ci * CK
                for t in range(CK // L):
                    d16 = dv[pl.ds(eb + t * L, L)]
                    r16 = rv[pl.ds(eb + t * L, L)]
                    ci2[ci, pl.ds(t * L, L)] = d16 * R + r16
                return c2
            lax.fori_loop(0, CJ, build_row, None)

            def scat(ci, c2):
                pltpu.sync_copy(ones, cs.at[ci2.at[ci]], add=True)
                return c2
            lax.fori_loop(0, CJ, scat, None)
            return carry
        lax.fori_loop(0, CW // SK, count_super, None)
        plsc.subcore_barrier()

        pltpu.sync_copy(cs, cl)

        def w_super(j, carry):
            base = wid * EW + j * SK
            pltpu.sync_copy(dst_hbm.at[pl.ds(base, SK)], dv)
            pltpu.sync_copy(rel_hbm.at[pl.ds(base, SK)], rv)

            def w_group(i, c2):
                off = i * L
                cidx = dv[pl.ds(off, L)] * R + rv[pl.ds(off, L)]
                cvals = plsc.load_gather(cl, [cidx])
                wv[pl.ds(off, L)] = 1.0 / jnp.maximum(cvals, 1.0)
                return c2
            lax.fori_loop(0, SK // L, w_group, None)
            pltpu.sync_copy(wv, w_hbm.at[pl.ds(base, SK)])
            return carry
        lax.fori_loop(0, EW // SK, w_super, None)

    return sc_edge_weights


@functools.cache
def _make_sc_aggregate():
    @functools.partial(
        pl.kernel,
        out_type=jax.ShapeDtypeStruct((NC, N, H), jnp.float32),
        mesh=_mesh(),
        scratch_types=[
            pltpu.VMEM_SHARED((N, H), jnp.float32),
            pltpu.VMEM((40, H), jnp.float32),
            pltpu.VMEM((SK,), jnp.int32),
            pltpu.VMEM((SK,), jnp.int32),
            pltpu.VMEM((CJ, CK), jnp.int32),
            pltpu.VMEM((SK,), jnp.float32),
            pltpu.VMEM((3, CK, H), jnp.float32),
        ] + [pltpu.SemaphoreType.DMA] * 6,
        compiler_params=pltpu.CompilerParams(needs_layout_passes=False),
    )
    def sc_aggregate(y_hbm, gidx_hbm, dst_hbm, w_hbm, part_hbm,
                     acc, zr, gv, dv, d2, wv, rows, *sems):
        gsem = sems[:3]
        ssem = sems[3:]
        cid = lax.axis_index("c")
        sid = lax.axis_index("s")
        wid = sid * NC + cid

        zeros = jnp.zeros((L,), jnp.float32)

        def zrow(i, carry):
            for t in range(H // L):
                zr[i, pl.ds(t * L, L)] = zeros
            return carry
        lax.fori_loop(0, 40, zrow, None)
        for k in range(15):
            pltpu.sync_copy(zr, acc.at[pl.ds(sid * 600 + k * 40, 40)])

        @pl.when(sid == 0)
        def _zero_tail():
            for k in range(10):
                pltpu.sync_copy(zr, acc.at[pl.ds(9600 + k * 40, 40)])
        plsc.subcore_barrier()

        HK = CK // 2

        def _gather(ci, b):
            pltpu.async_copy(y_hbm.at[gv.at[pl.ds(ci * CK, HK)]],
                             rows.at[b].at[pl.ds(0, HK)], gsem[b])
            pltpu.async_copy(y_hbm.at[gv.at[pl.ds(ci * CK + HK, HK)]],
                             rows.at[b].at[pl.ds(HK, HK)], gsem[b])

        def _gwait(b):
            pltpu.make_async_copy(y_hbm.at[pl.ds(0, CK)], rows.at[b],
                                  gsem[b]).wait()

        def _swait(b):
            pltpu.make_async_copy(y_hbm.at[pl.ds(0, CK)], rows.at[b],
                                  ssem[b]).wait()

        def _scale(ci, b):
            def body(g, carry):
                w16 = wv[pl.ds(ci * CK + g * L, L)]
                for j in range(L):
                    wb = jnp.full((L,), w16[j])
                    r = g * L + j
                    for h in range(H // L):
                        rows[b, r, pl.ds(h * L, L)] = (
                            rows[b, r, pl.ds(h * L, L)] * wb)
                return carry
            lax.fori_loop(0, CK // L, body, None)

        def super_chunk(j, carry):
            base = wid * EW + j * SK
            pltpu.sync_copy(gidx_hbm.at[pl.ds(base, SK)], gv)
            pltpu.sync_copy(w_hbm.at[pl.ds(base, SK)], wv)
            pltpu.sync_copy(dst_hbm.at[pl.ds(base, SK)], dv)

            def build_row(r, c2):
                eb = r * CK
                for t in range(CK // L):
                    d2[r, pl.ds(t * L, L)] = dv[pl.ds(eb + t * L, L)]
                return c2
            lax.fori_loop(0, CJ, build_row, None)

            _gather(0, 0)
            _gather(1, 1)

            def pipelined(t, c2):
                for b in range(3):
                    ci = t * 3 + b
                    bp = (b + 2) % 3
                    _gwait(b)
                    _scale(ci, b)
                    pltpu.async_copy(rows.at[b], acc.at[d2.at[ci]], ssem[b],
                                     add=True)

                    @pl.when(ci >= 1)
                    def _drain():
                        _swait(bp)

                    @pl.when(ci <= CJ - 3)
                    def _prefetch():
                        _gather(ci + 2, bp)
                return c2
            lax.fori_loop(0, (CJ - 1) // 3, pipelined, None)
            _gwait(0)
            _scale(CJ - 1, 0)
            pltpu.async_copy(rows.at[0], acc.at[d2.at[CJ - 1]], ssem[0],
                             add=True)
            _swait(2)
            _swait(0)
            return carry
        lax.fori_loop(0, EW // SK, super_chunk, None)
        plsc.subcore_barrier()

        for k in range(3):
            s = sid * 600 + k * 200
            pltpu.sync_copy(acc.at[pl.ds(s, 200)],
                            part_hbm.at[cid].at[pl.ds(s, 200)])

        @pl.when(sid == 0)
        def _copy_tail():
            for k in range(2):
                s = 9600 + k * 200
                pltpu.sync_copy(acc.at[pl.ds(s, 200)],
                                part_hbm.at[cid].at[pl.ds(s, 200)])

    return sc_aggregate


def _wcat_body(comp_ref, bases_ref, w_ref):
    w_ref[...] = jnp.dot(comp_ref[...], bases_ref[...],
                         preferred_element_type=jnp.float32)


def _tc_weights(comp, bases_flat):
    return pl.pallas_call(
        _wcat_body,
        out_shape=jax.ShapeDtypeStruct((R, H * H), jnp.float32),
    )(comp, bases_flat)


BM = 400


def _transform_body(x_ref, w_ref, root_ref, y_ref, self_ref):
    xb = x_ref[...]
    for r in range(R):
        y_ref[:, r, :] = jnp.dot(xb, w_ref[r], preferred_element_type=jnp.float32)
    self_ref[...] = jnp.dot(xb, root_ref[...], preferred_element_type=jnp.float32)


def _tc_transform(x, w3, root):
    return pl.pallas_call(
        _transform_body,
        grid=(N // BM,),
        in_specs=[
            pl.BlockSpec((BM, H), lambda i: (i, 0)),
            pl.BlockSpec((R, H, H), lambda i: (0, 0, 0)),
            pl.BlockSpec((H, H), lambda i: (0, 0)),
        ],
        out_specs=[
            pl.BlockSpec((BM, R, H), lambda i: (i, 0, 0)),
            pl.BlockSpec((BM, H), lambda i: (i, 0)),
        ],
        out_shape=[
            jax.ShapeDtypeStruct((N, R, H), jnp.float32),
            jax.ShapeDtypeStruct((N, H), jnp.float32),
        ],
    )(x, w3, root)


def _transform2_body(self_ref, p0_ref, p1_ref, g_ref, b_ref, w_ref, root_ref,
                     y_ref, self2_ref, h_ref):
    s = self_ref[...] + p0_ref[...] + p1_ref[...]
    hb = jnp.maximum(_ln(s, g_ref[...], b_ref[...]), 0.0)
    h_ref[...] = hb
    for r in range(R):
        y_ref[:, r, :] = jnp.dot(hb, w_ref[r], preferred_element_type=jnp.float32)
    self2_ref[...] = jnp.dot(hb, root_ref[...], preferred_element_type=jnp.float32)


def _tc_transform2(self1, p0, p1, gamma, beta, w3, root):
    blk = pl.BlockSpec((BM, H), lambda i: (i, 0))
    vec = pl.BlockSpec((1, H), lambda i: (0, 0))
    return pl.pallas_call(
        _transform2_body,
        grid=(N // BM,),
        in_specs=[blk, blk, blk, vec, vec,
                  pl.BlockSpec((R, H, H), lambda i: (0, 0, 0)),
                  pl.BlockSpec((H, H), lambda i: (0, 0))],
        out_specs=[pl.BlockSpec((BM, R, H), lambda i: (i, 0, 0)), blk, blk],
        out_shape=[
            jax.ShapeDtypeStruct((N, R, H), jnp.float32),
            jax.ShapeDtypeStruct((N, H), jnp.float32),
            jax.ShapeDtypeStruct((N, H), jnp.float32),
        ],
    )(self1, p0, p1, gamma.reshape(1, H), beta.reshape(1, H), w3, root)


def _ln(s, g, b):
    mu = jnp.mean(s, axis=-1, keepdims=True)
    var = jnp.mean((s - mu) ** 2, axis=-1, keepdims=True)
    return (s - mu) * lax.rsqrt(var + 1e-5) * g + b


def _combine1_body(self_ref, p0_ref, p1_ref, g_ref, b_ref, out_ref):
    s = self_ref[...] + p0_ref[...] + p1_ref[...]
    out_ref[...] = jnp.maximum(_ln(s, g_ref[...], b_ref[...]), 0.0)


def _combine2_body(self_ref, p0_ref, p1_ref, g_ref, b_ref, h_ref, rs_ref,
                   out_ref):
    s = self_ref[...] + p0_ref[...] + p1_ref[...]
    out_ref[...] = _ln(s, g_ref[...], b_ref[...]) + rs_ref[0, 0] * h_ref[...]


def _tc_combine1(self1, p0, p1, gamma, beta):
    blk = pl.BlockSpec((BM, H), lambda i: (i, 0))
    vec = pl.BlockSpec((1, H), lambda i: (0, 0))
    return pl.pallas_call(
        _combine1_body,
        grid=(N // BM,),
        in_specs=[blk, blk, blk, vec, vec],
        out_specs=blk,
        out_shape=jax.ShapeDtypeStruct((N, H), jnp.float32),
    )(self1, p0, p1, gamma.reshape(1, H), beta.reshape(1, H))


def _tc_combine2(self2, p0, p1, gamma, beta, h, res_scale):
    blk = pl.BlockSpec((BM, H), lambda i: (i, 0))
    vec = pl.BlockSpec((1, H), lambda i: (0, 0))
    sca = pl.BlockSpec((1, 1), lambda i: (0, 0))
    return pl.pallas_call(
        _combine2_body,
        grid=(N // BM,),
        in_specs=[blk, blk, blk, vec, vec, blk, sca],
        out_specs=blk,
        out_shape=jax.ShapeDtypeStruct((N, H), jnp.float32),
    )(self2, p0, p1, gamma.reshape(1, H), beta.reshape(1, H), h,
      jnp.asarray(res_scale, jnp.float32).reshape(1, 1))


def _layer(x, w3, root, gidx, dst, w_edge):
    y, self_term = _tc_transform(x, w3, root)
    part = _make_sc_aggregate()(y.reshape(N * R, H), gidx, dst, w_edge)
    return self_term, part[0], part[1]


def kernel(x, edge_index, edge_type, bases1, comp1, root1, gamma1, beta1,
           bases2, comp2, root2, gamma2, beta2, res_scale):
    src = edge_index[0].astype(jnp.int32)
    dst = edge_index[1].astype(jnp.int32)
    rel = edge_type.astype(jnp.int32)
    gidx = src * R + rel

    w_edge = _make_sc_edge_weights()(dst, rel)

    w31 = _tc_weights(comp1, bases1.reshape(NB, H * H)).reshape(R, H, H)
    s1, p10, p11 = _layer(x, w31, root1, gidx, dst, w_edge)

    w32 = _tc_weights(comp2, bases2.reshape(NB, H * H)).reshape(R, H, H)
    y2, s2, h = _tc_transform2(s1, p10, p11, gamma1, beta1, w32, root2)
    part2 = _make_sc_aggregate()(y2.reshape(N * R, H), gidx, dst, w_edge)
    return _tc_combine2(s2, part2[0], part2[1], gamma2, beta2, h, res_scale)


# speedup vs baseline: 1.0346x; 1.0346x over previous
import functools

import jax
import jax.numpy as jnp
from jax import lax
from jax.experimental import pallas as pl
from jax.experimental.pallas import tpu as pltpu
from jax.experimental.pallas import tpu_sc as plsc

N = 10000
R = 8
H = 128
NB = 6
NE = 320000

NC = 2
NS = 16
L = 16

EW = NE // (NC * NS)
CW = NE // NS
SK = 2000
CK = 80
CJ = SK // CK


def _mesh():
    return plsc.VectorSubcoreMesh(core_axis_name="c", subcore_axis_name="s",
                                  num_cores=NC, num_subcores=NS)


@functools.cache
def _make_sc_edge_weights():
    @functools.partial(
        pl.kernel,
        out_type=jax.ShapeDtypeStruct((NE,), jnp.float32),
        mesh=_mesh(),
        scratch_types=[
            pltpu.VMEM_SHARED((N * R,), jnp.float32),
            pltpu.VMEM((N * R,), jnp.float32),
            pltpu.VMEM((SK,), jnp.int32),
            pltpu.VMEM((SK,), jnp.int32),
            pltpu.VMEM((CJ, CK), jnp.int32),
            pltpu.VMEM((CK,), jnp.float32),
            pltpu.VMEM((SK,), jnp.float32),
            pltpu.VMEM((5008,), jnp.float32),
            pltpu.SemaphoreType.DMA,
        ],
        compiler_params=pltpu.CompilerParams(needs_layout_passes=False),
    )
    def sc_edge_weights(dst_hbm, rel_hbm, w_hbm, cs, cl, dv, rv, ci2, ones,
                        wv, zb, csem):
        cid = lax.axis_index("c")
        sid = lax.axis_index("s")
        wid = sid * NC + cid

        for t in range(CK // L):
            ones[pl.ds(t * L, L)] = jnp.ones((L,), jnp.float32)
        zeros = jnp.zeros((L,), jnp.float32)

        def zfill(i, carry):
            zb[pl.ds(i * L, L)] = zeros
            return carry
        lax.fori_loop(0, 313, zfill, None)
        zn = N * R // NS
        pltpu.sync_copy(zb.at[pl.ds(0, zn)], cs.at[pl.ds(sid * zn, zn)])
        plsc.subcore_barrier()

        def count_super(j, carry):
            base = sid * CW + j * SK
            pltpu.sync_copy(dst_hbm.at[pl.ds(base, SK)], dv)
            pltpu.sync_copy(rel_hbm.at[pl.ds(base, SK)], rv)

            def build_row(ci, c2):
                eb = ci * CK
                for t in range(CK // L):
                    d16 = dv[pl.ds(eb + t * L, L)]
                    r16 = rv[pl.ds(eb + t * L, L)]
                    ci2[ci, pl.ds(t * L, L)] = d16 * R + r16
                return c2
            lax.fori_loop(0, CJ, build_row, None)

            def scat(ci, c2):
                pltpu.async_copy(ones, cs.at[ci2.at[ci]], csem, add=True)
                return c2
            lax.fori_loop(0, CJ, scat, None)

            def sdrain(ci, c2):
                pltpu.make_async_copy(w_hbm.at[pl.ds(0, CK)], ones,
                                      csem).wait()
                return c2
            lax.fori_loop(0, CJ, sdrain, None)
            return carry
        lax.fori_loop(0, CW // SK, count_super, None)
        plsc.subcore_barrier()

        pltpu.sync_copy(cs, cl)

        def w_super(j, carry):
            base = wid * EW + j * SK
            pltpu.sync_copy(dst_hbm.at[pl.ds(base, SK)], dv)
            pltpu.sync_copy(rel_hbm.at[pl.ds(base, SK)], rv)

            def w_group(i, c2):
                off = i * L
                cidx = dv[pl.ds(off, L)] * R + rv[pl.ds(off, L)]
                cvals = plsc.load_gather(cl, [cidx])
                wv[pl.ds(off, L)] = 1.0 / jnp.maximum(cvals, 1.0)
                return c2
            lax.fori_loop(0, SK // L, w_group, None)
            pltpu.sync_copy(wv, w_hbm.at[pl.ds(base, SK)])
            return carry
        lax.fori_loop(0, EW // SK, w_super, None)

    return sc_edge_weights


@functools.cache
def _make_sc_aggregate():
    @functools.partial(
        pl.kernel,
        out_type=jax.ShapeDtypeStruct((NC, N, H), jnp.float32),
        mesh=_mesh(),
        scratch_types=[
            pltpu.VMEM_SHARED((N, H), jnp.float32),
            pltpu.VMEM((40, H), jnp.float32),
            pltpu.VMEM((SK,), jnp.int32),
            pltpu.VMEM((SK,), jnp.int32),
            pltpu.VMEM((CJ, CK), jnp.int32),
            pltpu.VMEM((SK,), jnp.float32),
            pltpu.VMEM((3, CK, H), jnp.float32),
        ] + [pltpu.SemaphoreType.DMA] * 6,
        compiler_params=pltpu.CompilerParams(needs_layout_passes=False),
    )
    def sc_aggregate(y_hbm, gidx_hbm, dst_hbm, w_hbm, part_hbm,
                     acc, zr, gv, dv, d2, wv, rows, *sems):
        gsem = sems[:3]
        ssem = sems[3:]
        cid = lax.axis_index("c")
        sid = lax.axis_index("s")
        wid = sid * NC + cid

        zeros = jnp.zeros((L,), jnp.float32)

        def zrow(i, carry):
            for t in range(H // L):
                zr[i, pl.ds(t * L, L)] = zeros
            return carry
        lax.fori_loop(0, 40, zrow, None)
        for k in range(15):
            pltpu.sync_copy(zr, acc.at[pl.ds(sid * 600 + k * 40, 40)])

        @pl.when(sid == 0)
        def _zero_tail():
            for k in range(10):
                pltpu.sync_copy(zr, acc.at[pl.ds(9600 + k * 40, 40)])
        plsc.subcore_barrier()

        def _gather(ci, b):
            pltpu.async_copy(y_hbm.at[gv.at[pl.ds(ci * CK, CK)]],
                             rows.at[b], gsem[b])

        def _gwait(b):
            pltpu.make_async_copy(y_hbm.at[pl.ds(0, CK)], rows.at[b],
                                  gsem[b]).wait()

        def _swait(b):
            pltpu.make_async_copy(y_hbm.at[pl.ds(0, CK)], rows.at[b],
                                  ssem[b]).wait()

        def _scale(ci, b):
            def body(g, carry):
                w16 = wv[pl.ds(ci * CK + g * L, L)]
                for j in range(L):
                    wb = jnp.full((L,), w16[j])
                    r = g * L + j
                    for h in range(H // L):
                        rows[b, r, pl.ds(h * L, L)] = (
                            rows[b, r, pl.ds(h * L, L)] * wb)
                return carry
            lax.fori_loop(0, CK // L, body, None)

        def super_chunk(j, carry):
            base = wid * EW + j * SK
            pltpu.sync_copy(gidx_hbm.at[pl.ds(base, SK)], gv)
            pltpu.sync_copy(w_hbm.at[pl.ds(base, SK)], wv)
            pltpu.sync_copy(dst_hbm.at[pl.ds(base, SK)], dv)

            def build_row(r, c2):
                eb = r * CK
                for t in range(CK // L):
                    d2[r, pl.ds(t * L, L)] = dv[pl.ds(eb + t * L, L)]
                return c2
            lax.fori_loop(0, CJ, build_row, None)

            _gather(0, 0)
            _gather(1, 1)

            def pipelined(t, c2):
                for b in range(3):
                    ci = t * 3 + b
                    bp = (b + 2) % 3
                    _gwait(b)
                    _scale(ci, b)
                    pltpu.async_copy(rows.at[b], acc.at[d2.at[ci]], ssem[b],
                                     add=True)

                    @pl.when(ci >= 1)
                    def _drain():
                        _swait(bp)

                    @pl.when(ci <= CJ - 3)
                    def _prefetch():
                        _gather(ci + 2, bp)
                return c2
            lax.fori_loop(0, (CJ - 1) // 3, pipelined, None)
            _gwait(0)
            _scale(CJ - 1, 0)
            pltpu.async_copy(rows.at[0], acc.at[d2.at[CJ - 1]], ssem[0],
                             add=True)
            _swait(2)
            _swait(0)
            return carry
        lax.fori_loop(0, EW // SK, super_chunk, None)
        plsc.subcore_barrier()

        for k in range(3):
            s = sid * 600 + k * 200
            pltpu.sync_copy(acc.at[pl.ds(s, 200)],
                            part_hbm.at[cid].at[pl.ds(s, 200)])

        @pl.when(sid == 0)
        def _copy_tail():
            for k in range(2):
                s = 9600 + k * 200
                pltpu.sync_copy(acc.at[pl.ds(s, 200)],
                                part_hbm.at[cid].at[pl.ds(s, 200)])

    return sc_aggregate


def _wcat_body(comp_ref, bases_ref, w_ref):
    w_ref[...] = jnp.dot(comp_ref[...], bases_ref[...],
                         preferred_element_type=jnp.float32)


def _tc_weights(comp, bases_flat):
    return pl.pallas_call(
        _wcat_body,
        out_shape=jax.ShapeDtypeStruct((R, H * H), jnp.float32),
    )(comp, bases_flat)


BM = 400


def _transform_body(x_ref, w_ref, root_ref, y_ref, self_ref):
    xb = x_ref[...]
    for r in range(R):
        y_ref[:, r, :] = jnp.dot(xb, w_ref[r], preferred_element_type=jnp.float32)
    self_ref[...] = jnp.dot(xb, root_ref[...], preferred_element_type=jnp.float32)


def _tc_transform(x, w3, root):
    return pl.pallas_call(
        _transform_body,
        grid=(N // BM,),
        in_specs=[
            pl.BlockSpec((BM, H), lambda i: (i, 0)),
            pl.BlockSpec((R, H, H), lambda i: (0, 0, 0)),
            pl.BlockSpec((H, H), lambda i: (0, 0)),
        ],
        out_specs=[
            pl.BlockSpec((BM, R, H), lambda i: (i, 0, 0)),
            pl.BlockSpec((BM, H), lambda i: (i, 0)),
        ],
        out_shape=[
            jax.ShapeDtypeStruct((N, R, H), jnp.float32),
            jax.ShapeDtypeStruct((N, H), jnp.float32),
        ],
    )(x, w3, root)


def _transform2_body(self_ref, p0_ref, p1_ref, g_ref, b_ref, w_ref, root_ref,
                     y_ref, self2_ref, h_ref):
    s = self_ref[...] + p0_ref[...] + p1_ref[...]
    hb = jnp.maximum(_ln(s, g_ref[...], b_ref[...]), 0.0)
    h_ref[...] = hb
    for r in range(R):
        y_ref[:, r, :] = jnp.dot(hb, w_ref[r], preferred_element_type=jnp.float32)
    self2_ref[...] = jnp.dot(hb, root_ref[...], preferred_element_type=jnp.float32)


def _tc_transform2(self1, p0, p1, gamma, beta, w3, root):
    blk = pl.BlockSpec((BM, H), lambda i: (i, 0))
    vec = pl.BlockSpec((1, H), lambda i: (0, 0))
    return pl.pallas_call(
        _transform2_body,
        grid=(N // BM,),
        in_specs=[blk, blk, blk, vec, vec,
                  pl.BlockSpec((R, H, H), lambda i: (0, 0, 0)),
                  pl.BlockSpec((H, H), lambda i: (0, 0))],
        out_specs=[pl.BlockSpec((BM, R, H), lambda i: (i, 0, 0)), blk, blk],
        out_shape=[
            jax.ShapeDtypeStruct((N, R, H), jnp.float32),
            jax.ShapeDtypeStruct((N, H), jnp.float32),
            jax.ShapeDtypeStruct((N, H), jnp.float32),
        ],
    )(self1, p0, p1, gamma.reshape(1, H), beta.reshape(1, H), w3, root)


def _ln(s, g, b):
    mu = jnp.mean(s, axis=-1, keepdims=True)
    var = jnp.mean((s - mu) ** 2, axis=-1, keepdims=True)
    return (s - mu) * lax.rsqrt(var + 1e-5) * g + b


def _combine1_body(self_ref, p0_ref, p1_ref, g_ref, b_ref, out_ref):
    s = self_ref[...] + p0_ref[...] + p1_ref[...]
    out_ref[...] = jnp.maximum(_ln(s, g_ref[...], b_ref[...]), 0.0)


def _combine2_body(self_ref, p0_ref, p1_ref, g_ref, b_ref, h_ref, rs_ref,
                   out_ref):
    s = self_ref[...] + p0_ref[...] + p1_ref[...]
    out_ref[...] = _ln(s, g_ref[...], b_ref[...]) + rs_ref[0, 0] * h_ref[...]


def _tc_combine1(self1, p0, p1, gamma, beta):
    blk = pl.BlockSpec((BM, H), lambda i: (i, 0))
    vec = pl.BlockSpec((1, H), lambda i: (0, 0))
    return pl.pallas_call(
        _combine1_body,
        grid=(N // BM,),
        in_specs=[blk, blk, blk, vec, vec],
        out_specs=blk,
        out_shape=jax.ShapeDtypeStruct((N, H), jnp.float32),
    )(self1, p0, p1, gamma.reshape(1, H), beta.reshape(1, H))


def _tc_combine2(self2, p0, p1, gamma, beta, h, res_scale):
    blk = pl.BlockSpec((BM, H), lambda i: (i, 0))
    vec = pl.BlockSpec((1, H), lambda i: (0, 0))
    sca = pl.BlockSpec((1, 1), lambda i: (0, 0))
    return pl.pallas_call(
        _combine2_body,
        grid=(N // BM,),
        in_specs=[blk, blk, blk, vec, vec, blk, sca],
        out_specs=blk,
        out_shape=jax.ShapeDtypeStruct((N, H), jnp.float32),
    )(self2, p0, p1, gamma.reshape(1, H), beta.reshape(1, H), h,
      jnp.asarray(res_scale, jnp.float32).reshape(1, 1))


def _layer(x, w3, root, gidx, dst, w_edge):
    y, self_term = _tc_transform(x, w3, root)
    part = _make_sc_aggregate()(y.reshape(N * R, H), gidx, dst, w_edge)
    return self_term, part[0], part[1]


def kernel(x, edge_index, edge_type, bases1, comp1, root1, gamma1, beta1,
           bases2, comp2, root2, gamma2, beta2, res_scale):
    src = edge_index[0].astype(jnp.int32)
    dst = edge_index[1].astype(jnp.int32)
    rel = edge_type.astype(jnp.int32)
    gidx = src * R + rel

    w_edge = _make_sc_edge_weights()(dst, rel)

    w31 = _tc_weights(comp1, bases1.reshape(NB, H * H)).reshape(R, H, H)
    s1, p10, p11 = _layer(x, w31, root1, gidx, dst, w_edge)

    w32 = _tc_weights(comp2, bases2.reshape(NB, H * H)).reshape(R, H, H)
    y2, s2, h = _tc_transform2(s1, p10, p11, gamma1, beta1, w32, root2)
    part2 = _make_sc_aggregate()(y2.reshape(N * R, H), gidx, dst, w_edge)
    return _tc_combine2(s2, part2[0], part2[1], gamma2, beta2, h, res_scale)
